# Initial kernel scaffold; baseline (speedup 1.0000x reference)
import jax
import jax.numpy as jnp
from jax.experimental import pallas as pl


def kernel(x, edge_index, W1, b1, g1, be1, a1, W2, b2, g2, be2, a2):
    raise NotImplementedError("write your pallas kernel here")


# speedup vs baseline: 11.9039x; 11.9039x over previous
import functools

import jax
import jax.numpy as jnp
from jax import lax
from jax.experimental import pallas as pl
from jax.experimental.pallas import tpu as pltpu
from jax.experimental.pallas import tpu_sc as plsc

N = 10000
E = 320000
D = 128

NC = 2
NS = 16
NW = NC * NS
EPT = E // NW
K = 80
CH = EPT // K
NP = 10240
RPT = NP // NS
DEGW = 128

_MESH = plsc.VectorSubcoreMesh(core_axis_name="c", subcore_axis_name="s")


@functools.partial(
    pl.kernel,
    out_type=jax.ShapeDtypeStruct((NC * NP, DEGW), jnp.float32),
    mesh=_MESH,
    scratch_types=[
        pltpu.VMEM((K,), jnp.int32),
        pltpu.VMEM((K, DEGW), jnp.float32),
        pltpu.VMEM_SHARED((NP, DEGW), jnp.float32),
    ],
)
def _deg_call(dst_hbm, ones_hbm, zeros_hbm, out_hbm, idx_v, ones_v, acc_sh):
  cid = lax.axis_index("c")
  sid = lax.axis_index("s")
  wid = sid * NC + cid
  rbase = pl.multiple_of(sid * RPT, 8)
  obase = pl.multiple_of(cid * NP + sid * RPT, 8)

  pltpu.sync_copy(ones_hbm, ones_v)
  pltpu.sync_copy(zeros_hbm.at[pl.ds(rbase, RPT)], acc_sh.at[pl.ds(rbase, RPT)])
  plsc.subcore_barrier()

  def chunk(j, carry):
    base = pl.multiple_of(wid * EPT + j * K, 8)
    pltpu.sync_copy(dst_hbm.at[pl.ds(base, K)], idx_v)
    pltpu.sync_copy(ones_v, acc_sh.at[idx_v], add=True)
    return carry

  lax.fori_loop(0, CH, chunk, 0)
  plsc.subcore_barrier()

  pltpu.sync_copy(acc_sh.at[pl.ds(rbase, RPT)], out_hbm.at[pl.ds(obase, RPT)])


@functools.partial(
    pl.kernel,
    out_type=jax.ShapeDtypeStruct((NC * NP, D), jnp.float32),
    mesh=_MESH,
    scratch_types=[
        pltpu.VMEM((K,), jnp.int32),
        pltpu.VMEM((K,), jnp.int32),
        pltpu.VMEM((K, D), jnp.float32),
        pltpu.VMEM_SHARED((NP, D), jnp.float32),
        pltpu.SemaphoreType.DMA,
    ],
)
def _msg_call(hp_hbm, src_hbm, dst_hbm, zeros_hbm, out_hbm,
              sidx_v, didx_v, rows_v, acc_sh, gsem):
  cid = lax.axis_index("c")
  sid = lax.axis_index("s")
  wid = sid * NC + cid
  rbase = pl.multiple_of(sid * RPT, 8)
  obase = pl.multiple_of(cid * NP + sid * RPT, 8)

  @pl.when(cid == 0)
  def _():
    pltpu.sync_copy(hp_hbm.at[pl.ds(rbase, RPT)], acc_sh.at[pl.ds(rbase, RPT)])

  @pl.when(cid != 0)
  def _():
    pltpu.sync_copy(zeros_hbm.at[pl.ds(rbase, RPT)],
                    acc_sh.at[pl.ds(rbase, RPT)])

  plsc.subcore_barrier()

  def chunk(j, carry):
    base = pl.multiple_of(wid * EPT + j * K, 8)
    pltpu.sync_copy(src_hbm.at[pl.ds(base, K)], sidx_v)
    pltpu.sync_copy(dst_hbm.at[pl.ds(base, K)], didx_v)
    pltpu.async_copy(hp_hbm.at[sidx_v], rows_v, gsem).wait()
    pltpu.sync_copy(rows_v, acc_sh.at[didx_v], add=True)
    return carry

  lax.fori_loop(0, CH, chunk, 0)
  plsc.subcore_barrier()

  pltpu.sync_copy(acc_sh.at[pl.ds(rbase, RPT)], out_hbm.at[pl.ds(obase, RPT)])


BLK = 1280
GRID = NP // BLK

_row_spec = pl.BlockSpec((BLK, D), lambda i: (i, 0))
_deg_spec = pl.BlockSpec((BLK, DEGW), lambda i: (i, 0))
_mat_spec = pl.BlockSpec((D, D), lambda i: (0, 0))
_vec_spec = pl.BlockSpec((1, D), lambda i: (0, 0))
_scl_spec = pl.BlockSpec((1, 1), lambda i: (0, 0))


def _dinv(deg0_ref, deg1_ref):
  deg = deg0_ref[:, :1] + deg1_ref[:, :1] + 1.0
  return lax.rsqrt(deg)


def _pre_body(x_ref, w_ref, deg0_ref, deg1_ref, out_ref):
  h = jnp.dot(x_ref[...], w_ref[...], preferred_element_type=jnp.float32)
  out_ref[...] = _dinv(deg0_ref, deg1_ref) * h


def _ln_prelu(z, g_ref, be_ref, a_ref):
  mu = jnp.mean(z, axis=-1, keepdims=True)
  var = jnp.mean((z - mu) ** 2, axis=-1, keepdims=True)
  zn = (z - mu) * lax.rsqrt(var + 1e-5) * g_ref[...] + be_ref[...]
  return jnp.where(zn >= 0, zn, a_ref[0, 0] * zn)


def _mid_body(p0_ref, p1_ref, deg0_ref, deg1_ref, b_ref, g_ref, be_ref,
              a_ref, w_ref, out_ref):
  dinv = _dinv(deg0_ref, deg1_ref)
  z = dinv * (p0_ref[...] + p1_ref[...]) + b_ref[...]
  y = _ln_prelu(z, g_ref, be_ref, a_ref)
  out_ref[...] = dinv * jnp.dot(y, w_ref[...],
                                preferred_element_type=jnp.float32)


def _fin_body(p0_ref, p1_ref, deg0_ref, deg1_ref, b_ref, g_ref, be_ref,
              a_ref, out_ref):
  dinv = _dinv(deg0_ref, deg1_ref)
  z = dinv * (p0_ref[...] + p1_ref[...]) + b_ref[...]
  out_ref[...] = _ln_prelu(z, g_ref, be_ref, a_ref)


_pre_call = pl.pallas_call(
    _pre_body,
    grid=(GRID,),
    in_specs=[_row_spec, _mat_spec, _deg_spec, _deg_spec],
    out_specs=_row_spec,
    out_shape=jax.ShapeDtypeStruct((NP, D), jnp.float32),
)

_mid_call = pl.pallas_call(
    _mid_body,
    grid=(GRID,),
    in_specs=[_row_spec, _row_spec, _deg_spec, _deg_spec,
              _vec_spec, _vec_spec, _vec_spec, _scl_spec, _mat_spec],
    out_specs=_row_spec,
    out_shape=jax.ShapeDtypeStruct((NP, D), jnp.float32),
)

_fin_call = pl.pallas_call(
    _fin_body,
    grid=(GRID,),
    in_specs=[_row_spec, _row_spec, _deg_spec, _deg_spec,
              _vec_spec, _vec_spec, _vec_spec, _scl_spec],
    out_specs=_row_spec,
    out_shape=jax.ShapeDtypeStruct((NP, D), jnp.float32),
)


def kernel(x, edge_index, W1, b1, g1, be1, a1, W2, b2, g2, be2, a2):
  src = edge_index[0]
  dst = edge_index[1]
  xp = jnp.pad(x, ((0, NP - N), (0, 0)))
  ones_blk = jnp.ones((K, DEGW), jnp.float32)
  zeros16 = jnp.zeros((NP, DEGW), jnp.float32)
  zerosd = jnp.zeros((NP, D), jnp.float32)

  degp = _deg_call(dst, ones_blk, zeros16)
  deg0, deg1 = degp[:NP], degp[NP:]

  b1r, g1r, be1r = b1.reshape(1, D), g1.reshape(1, D), be1.reshape(1, D)
  b2r, g2r, be2r = b2.reshape(1, D), g2.reshape(1, D), be2.reshape(1, D)
  a1r, a2r = a1.reshape(1, 1), a2.reshape(1, 1)

  hp1 = _pre_call(xp, W1, deg0, deg1)
  parts1 = _msg_call(hp1, src, dst, zerosd)
  hp2 = _mid_call(parts1[:NP], parts1[NP:], deg0, deg1,
                  b1r, g1r, be1r, a1r, W2)
  parts2 = _msg_call(hp2, src, dst, zerosd)
  out = _fin_call(parts2[:NP], parts2[NP:], deg0, deg1,
                  b2r, g2r, be2r, a2r)
  return out[:N]


# speedup vs baseline: 20.2071x; 1.6975x over previous
import functools

import jax
import jax.numpy as jnp
from jax import lax
from jax.experimental import pallas as pl
from jax.experimental.pallas import tpu as pltpu
from jax.experimental.pallas import tpu_sc as plsc

N = 10000
E = 320000
D = 128

NC = 2
NS = 16
NW = NC * NS
EPT = E // NW
K = 80
CH = EPT // K
NB = 4
GROUPS = CH // NB
TAIL = CH - GROUPS * NB
NP = 10240
RPT = NP // NS
DEGW = 128

_MESH = plsc.VectorSubcoreMesh(core_axis_name="c", subcore_axis_name="s")


@functools.partial(
    pl.kernel,
    out_type=jax.ShapeDtypeStruct((NC * NP, DEGW), jnp.float32),
    mesh=_MESH,
    scratch_types=(
        [pltpu.VMEM((NB, K), jnp.int32)]
        + [pltpu.VMEM((K, DEGW), jnp.float32)]
        + [pltpu.VMEM_SHARED((NP, DEGW), jnp.float32)]
        + [pltpu.SemaphoreType.DMA] * NB
    ),
)
def _deg_call(dst_hbm, ones_hbm, zeros_hbm, out_hbm, *scr):
  idx_v = scr[0]
  ones_v = scr[1]
  acc_sh = scr[2]
  isem = scr[3:3 + NB]
  cid = lax.axis_index("c")
  sid = lax.axis_index("s")
  wid = sid * NC + cid
  rbase = pl.multiple_of(sid * RPT, 8)

  pltpu.sync_copy(ones_hbm, ones_v)
  pltpu.sync_copy(zeros_hbm.at[pl.ds(rbase, RPT)], acc_sh.at[pl.ds(rbase, RPT)])
  plsc.subcore_barrier()

  def group(g, carry):
    gbase = wid * EPT + g * (NB * K)
    idesc = []
    for b in range(NB):
      base = pl.multiple_of(gbase + b * K, 8)
      idesc.append(
          pltpu.async_copy(dst_hbm.at[pl.ds(base, K)], idx_v.at[b], isem[b]))
    for b in range(NB):
      idesc[b].wait()

    def scat(b, c):
      pltpu.sync_copy(ones_v, acc_sh.at[idx_v.at[b]], add=True)
      return c

    lax.fori_loop(0, NB, scat, 0)
    return carry

  lax.fori_loop(0, GROUPS, group, 0)
  for t in range(TAIL):
    base = pl.multiple_of(wid * EPT + (GROUPS * NB + t) * K, 8)
    pltpu.sync_copy(dst_hbm.at[pl.ds(base, K)], idx_v.at[0])
    pltpu.sync_copy(ones_v, acc_sh.at[idx_v.at[0]], add=True)
  plsc.subcore_barrier()

  obase = pl.multiple_of(cid * NP + sid * RPT, 8)
  pltpu.sync_copy(acc_sh.at[pl.ds(rbase, RPT)], out_hbm.at[pl.ds(obase, RPT)])


@functools.partial(
    pl.kernel,
    out_type=jax.ShapeDtypeStruct((NC * NP, D), jnp.float32),
    mesh=_MESH,
    scratch_types=(
        [pltpu.VMEM((NB * K,), jnp.int32)]
        + [pltpu.VMEM((NB, K), jnp.int32)]
        + [pltpu.VMEM((NB * K, D), jnp.float32)]
        + [pltpu.VMEM_SHARED((NP, D), jnp.float32)]
        + [pltpu.SemaphoreType.DMA] * (2 * NB)
    ),
)
def _msg_call(hp_hbm, src_hbm, dst_hbm, zeros_hbm, out_hbm, *scr):
  sidx = scr[0]
  didx = scr[1]
  rows = scr[2]
  acc_sh = scr[3]
  isem = scr[4:4 + NB]
  gsem = scr[4 + NB:4 + 2 * NB]
  cid = lax.axis_index("c")
  sid = lax.axis_index("s")
  wid = sid * NC + cid
  rbase = pl.multiple_of(sid * RPT, 8)

  @pl.when(cid == 0)
  def _():
    pltpu.sync_copy(hp_hbm.at[pl.ds(rbase, RPT)], acc_sh.at[pl.ds(rbase, RPT)])

  @pl.when(cid != 0)
  def _():
    pltpu.sync_copy(zeros_hbm.at[pl.ds(rbase, RPT)],
                    acc_sh.at[pl.ds(rbase, RPT)])

  plsc.subcore_barrier()

  def group(g, carry):
    gbase = wid * EPT + g * (NB * K)
    idesc = []
    for b in range(NB):
      base = pl.multiple_of(gbase + b * K, 8)
      idesc.append(
          pltpu.async_copy(src_hbm.at[pl.ds(base, K)],
                           sidx.at[pl.ds(b * K, K)], isem[b]))
      idesc.append(
          pltpu.async_copy(dst_hbm.at[pl.ds(base, K)], didx.at[b], isem[b]))
    gdesc = []
    for b in range(NB):
      idesc[2 * b].wait()
      gdesc.append(
          pltpu.async_copy(hp_hbm.at[sidx.at[pl.ds(b * K, K)]],
                           rows.at[pl.ds(b * K, K)], gsem[b]))
    for b in range(NB):
      gdesc[b].wait()
      idesc[2 * b + 1].wait()

    def scat(b, c):
      off = pl.multiple_of(b * K, 8)
      pltpu.sync_copy(rows.at[pl.ds(off, K)], acc_sh.at[didx.at[b]], add=True)
      return c

    lax.fori_loop(0, NB, scat, 0)
    return carry

  lax.fori_loop(0, GROUPS, group, 0)
  for t in range(TAIL):
    base = pl.multiple_of(wid * EPT + (GROUPS * NB + t) * K, 8)
    pltpu.sync_copy(src_hbm.at[pl.ds(base, K)], sidx.at[pl.ds(0, K)])
    pltpu.sync_copy(dst_hbm.at[pl.ds(base, K)], didx.at[0])
    pltpu.async_copy(hp_hbm.at[sidx.at[pl.ds(0, K)]],
                     rows.at[pl.ds(0, K)], gsem[0]).wait()
    pltpu.sync_copy(rows.at[pl.ds(0, K)], acc_sh.at[didx.at[0]], add=True)
  plsc.subcore_barrier()

  obase = pl.multiple_of(cid * NP + sid * RPT, 8)
  pltpu.sync_copy(acc_sh.at[pl.ds(rbase, RPT)], out_hbm.at[pl.ds(obase, RPT)])


BLK = 1280
GRID = NP // BLK
FBLK = BLK
FGRID = GRID
DW = 8

_row_spec = pl.BlockSpec((BLK, D), lambda i: (i, 0))
_row1_spec = pl.BlockSpec((BLK, D), lambda i: (i + GRID, 0))
_deg_spec = pl.BlockSpec((BLK, DEGW), lambda i: (i, 0))
_deg1_spec = pl.BlockSpec((BLK, DEGW), lambda i: (i + GRID, 0))
_dinv_spec = pl.BlockSpec((BLK, DW), lambda i: (i, 0))
_mat_spec = pl.BlockSpec((D, D), lambda i: (0, 0))
_vec_spec = pl.BlockSpec((1, D), lambda i: (0, 0))
_scl_spec = pl.BlockSpec((1, 1), lambda i: (0, 0))
_frow1_spec = pl.BlockSpec((FBLK, D), lambda i: (i + GRID, 0))


def _pre_body(x_ref, w_ref, deg0_ref, deg1_ref, hp_ref, dinv_ref):
  dinv = lax.rsqrt(deg0_ref[:, :1] + deg1_ref[:, :1] + 1.0)
  h = jnp.dot(x_ref[...], w_ref[...], preferred_element_type=jnp.float32)
  hp_ref[...] = dinv * h
  dinv_ref[...] = jnp.broadcast_to(dinv, (dinv.shape[0], DW))


def _ln_prelu(z, g_ref, be_ref, a_ref):
  mu = jnp.mean(z, axis=-1, keepdims=True)
  var = jnp.mean((z - mu) ** 2, axis=-1, keepdims=True)
  zn = (z - mu) * lax.rsqrt(var + 1e-5) * g_ref[...] + be_ref[...]
  return jnp.where(zn >= 0, zn, a_ref[0, 0] * zn)


def _mid_body(p0_ref, p1_ref, dinv_ref, b_ref, g_ref, be_ref,
              a_ref, w_ref, out_ref):
  dinv = dinv_ref[:, :1]
  z = dinv * (p0_ref[...] + p1_ref[...]) + b_ref[...]
  y = _ln_prelu(z, g_ref, be_ref, a_ref)
  out_ref[...] = dinv * jnp.dot(y, w_ref[...],
                                preferred_element_type=jnp.float32)


def _fin_body(p0_ref, p1_ref, dinv_ref, b_ref, g_ref, be_ref,
              a_ref, out_ref):
  dinv = dinv_ref[:, :1]
  z = dinv * (p0_ref[...] + p1_ref[...]) + b_ref[...]
  out_ref[...] = _ln_prelu(z, g_ref, be_ref, a_ref)


_pre_call = pl.pallas_call(
    _pre_body,
    grid=(GRID,),
    in_specs=[_row_spec, _mat_spec, _deg_spec, _deg1_spec],
    out_specs=[_row_spec, _dinv_spec],
    out_shape=[jax.ShapeDtypeStruct((NP, D), jnp.float32),
               jax.ShapeDtypeStruct((NP, DW), jnp.float32)],
)

_mid_call = pl.pallas_call(
    _mid_body,
    grid=(GRID,),
    in_specs=[_row_spec, _row1_spec, _dinv_spec,
              _vec_spec, _vec_spec, _vec_spec, _scl_spec, _mat_spec],
    out_specs=_row_spec,
    out_shape=jax.ShapeDtypeStruct((NP, D), jnp.float32),
)

_fin_call = pl.pallas_call(
    _fin_body,
    grid=(FGRID,),
    in_specs=[_row_spec, _frow1_spec, _dinv_spec,
              _vec_spec, _vec_spec, _vec_spec, _scl_spec],
    out_specs=_row_spec,
    out_shape=jax.ShapeDtypeStruct((NP, D), jnp.float32),
)


def kernel(x, edge_index, W1, b1, g1, be1, a1, W2, b2, g2, be2, a2):
  src = edge_index[0]
  dst = edge_index[1]
  ones_blk = jnp.ones((K, DEGW), jnp.float32)
  zeros_deg = jnp.zeros((NP, DEGW), jnp.float32)
  zerosd = jnp.zeros((NP, D), jnp.float32)

  degp = _deg_call(dst, ones_blk, zeros_deg)

  b1r, g1r, be1r = b1.reshape(1, D), g1.reshape(1, D), be1.reshape(1, D)
  b2r, g2r, be2r = b2.reshape(1, D), g2.reshape(1, D), be2.reshape(1, D)
  a1r, a2r = a1.reshape(1, 1), a2.reshape(1, 1)

  hp1, dinv8 = _pre_call(x, W1, degp, degp)
  parts1 = _msg_call(hp1, src, dst, zerosd)
  hp2 = _mid_call(parts1, parts1, dinv8, b1r, g1r, be1r, a1r, W2)
  parts2 = _msg_call(hp2, src, dst, zerosd)
  return _fin_call(parts2, parts2, dinv8, b2r, g2r, be2r, a2r)[:N]


# speedup vs baseline: 25.1927x; 1.2467x over previous
import functools

import jax
import jax.numpy as jnp
from jax import lax
from jax.experimental import pallas as pl
from jax.experimental.pallas import tpu as pltpu
from jax.experimental.pallas import tpu_sc as plsc

N = 10000
E = 320000
D = 128

NC = 2
NS = 16
NW = NC * NS
EPT = E // NW
K = 80
CH = EPT // K
NB = 4
GROUPS = CH // NB
TAIL = CH - GROUPS * NB
CPH = 2
NH = CH // CPH
MSG_TAIL = CH - NH * CPH
NP = 10240
RPT = NP // NS
DEGW = 128

_MESH = plsc.VectorSubcoreMesh(core_axis_name="c", subcore_axis_name="s")


@functools.partial(
    pl.kernel,
    out_type=jax.ShapeDtypeStruct((NC * NP, DEGW), jnp.float32),
    mesh=_MESH,
    scratch_types=(
        [pltpu.VMEM((NB, K), jnp.int32)]
        + [pltpu.VMEM((K, DEGW), jnp.float32)]
        + [pltpu.VMEM_SHARED((NP, DEGW), jnp.float32)]
        + [pltpu.SemaphoreType.DMA] * NB
    ),
)
def _deg_call(dst_hbm, ones_hbm, zeros_hbm, out_hbm, *scr):
  idx_v = scr[0]
  ones_v = scr[1]
  acc_sh = scr[2]
  isem = scr[3:3 + NB]
  cid = lax.axis_index("c")
  sid = lax.axis_index("s")
  wid = sid * NC + cid
  rbase = pl.multiple_of(sid * RPT, 8)

  pltpu.sync_copy(ones_hbm, ones_v)
  pltpu.sync_copy(zeros_hbm.at[pl.ds(rbase, RPT)], acc_sh.at[pl.ds(rbase, RPT)])
  plsc.subcore_barrier()

  def group(g, carry):
    gbase = wid * EPT + g * (NB * K)
    idesc = []
    for b in range(NB):
      base = pl.multiple_of(gbase + b * K, 8)
      idesc.append(
          pltpu.async_copy(dst_hbm.at[pl.ds(base, K)], idx_v.at[b], isem[b]))
    for b in range(NB):
      idesc[b].wait()

    def scat(b, c):
      pltpu.sync_copy(ones_v, acc_sh.at[idx_v.at[b]], add=True)
      return c

    lax.fori_loop(0, NB, scat, 0)
    return carry

  lax.fori_loop(0, GROUPS, group, 0)
  for t in range(TAIL):
    base = pl.multiple_of(wid * EPT + (GROUPS * NB + t) * K, 8)
    pltpu.sync_copy(dst_hbm.at[pl.ds(base, K)], idx_v.at[0])
    pltpu.sync_copy(ones_v, acc_sh.at[idx_v.at[0]], add=True)
  plsc.subcore_barrier()

  obase = pl.multiple_of(cid * NP + sid * RPT, 8)
  pltpu.sync_copy(acc_sh.at[pl.ds(rbase, RPT)], out_hbm.at[pl.ds(obase, RPT)])


@functools.partial(
    pl.kernel,
    out_type=jax.ShapeDtypeStruct((NC * NP, D), jnp.float32),
    mesh=_MESH,
    scratch_types=(
        [pltpu.VMEM((2 * CPH * K,), jnp.int32)]
        + [pltpu.VMEM((2 * CPH, K), jnp.int32)]
        + [pltpu.VMEM((2 * CPH * K, D), jnp.float32)]
        + [pltpu.VMEM_SHARED((NP, D), jnp.float32)]
        + [pltpu.SemaphoreType.DMA] * 4
    ),
)
def _msg_call(hp_hbm, src_hbm, dst_hbm, zeros_hbm, out_hbm, *scr):
  sidx = scr[0]
  didx = scr[1]
  rows = scr[2]
  acc_sh = scr[3]
  isem = scr[4:6]
  gsem = scr[6:8]
  cid = lax.axis_index("c")
  sid = lax.axis_index("s")
  wid = sid * NC + cid
  rbase = pl.multiple_of(sid * RPT, 8)

  @pl.when(cid == 0)
  def _():
    pltpu.sync_copy(hp_hbm.at[pl.ds(rbase, RPT)], acc_sh.at[pl.ds(rbase, RPT)])

  @pl.when(cid != 0)
  def _():
    pltpu.sync_copy(zeros_hbm.at[pl.ds(rbase, RPT)],
                    acc_sh.at[pl.ds(rbase, RPT)])

  plsc.subcore_barrier()

  def issue_half(h, slot):
    hbase = pl.multiple_of(wid * EPT + h * (CPH * K), 8)
    soff = slot * (CPH * K)
    descs = [pltpu.async_copy(src_hbm.at[pl.ds(hbase, CPH * K)],
                              sidx.at[pl.ds(soff, CPH * K)], isem[slot])]
    for b in range(CPH):
      base = pl.multiple_of(hbase + b * K, 8)
      descs.append(pltpu.async_copy(dst_hbm.at[pl.ds(base, K)],
                                    didx.at[CPH * slot + b], isem[slot]))
    for d in descs:
      d.wait()
    for b in range(CPH):
      pltpu.async_copy(hp_hbm.at[sidx.at[pl.ds(soff + b * K, K)]],
                       rows.at[pl.ds(soff + b * K, K)], gsem[slot])

  def drain_half(slot):
    for b in range(CPH):
      off = slot * (CPH * K) + b * K
      pltpu.make_async_copy(hp_hbm.at[sidx.at[pl.ds(off, K)]],
                            rows.at[pl.ds(off, K)], gsem[slot]).wait()

  issue_half(0, 0)

  def half_body(h, carry):
    slot = lax.rem(h, 2)

    @pl.when(jnp.logical_and(slot == 0, h + 1 < NH))
    def _():
      issue_half(h + 1, 1)

    @pl.when(jnp.logical_and(slot != 0, h + 1 < NH))
    def _():
      issue_half(h + 1, 0)

    @pl.when(slot == 0)
    def _():
      drain_half(0)

    @pl.when(slot != 0)
    def _():
      drain_half(1)

    def scat(b, c):
      off = pl.multiple_of(slot * (CPH * K) + b * K, 8)
      pltpu.sync_copy(rows.at[pl.ds(off, K)],
                      acc_sh.at[didx.at[CPH * slot + b]], add=True)
      return c

    lax.fori_loop(0, CPH, scat, 0)
    return carry

  lax.fori_loop(0, NH, half_body, 0)

  for t in range(MSG_TAIL):
    base = pl.multiple_of(wid * EPT + (NH * CPH + t) * K, 8)
    pltpu.sync_copy(src_hbm.at[pl.ds(base, K)], sidx.at[pl.ds(0, K)])
    pltpu.sync_copy(dst_hbm.at[pl.ds(base, K)], didx.at[0])
    pltpu.async_copy(hp_hbm.at[sidx.at[pl.ds(0, K)]],
                     rows.at[pl.ds(0, K)], gsem[0]).wait()
    pltpu.sync_copy(rows.at[pl.ds(0, K)], acc_sh.at[didx.at[0]], add=True)
  plsc.subcore_barrier()

  obase = pl.multiple_of(cid * NP + sid * RPT, 8)
  pltpu.sync_copy(acc_sh.at[pl.ds(rbase, RPT)], out_hbm.at[pl.ds(obase, RPT)])


BLK = 1280
GRID = NP // BLK
FBLK = BLK
FGRID = GRID
DW = 8

_row_spec = pl.BlockSpec((BLK, D), lambda i: (i, 0))
_row1_spec = pl.BlockSpec((BLK, D), lambda i: (i + GRID, 0))
_deg_spec = pl.BlockSpec((BLK, DEGW), lambda i: (i, 0))
_deg1_spec = pl.BlockSpec((BLK, DEGW), lambda i: (i + GRID, 0))
_dinv_spec = pl.BlockSpec((BLK, DW), lambda i: (i, 0))
_mat_spec = pl.BlockSpec((D, D), lambda i: (0, 0))
_vec_spec = pl.BlockSpec((1, D), lambda i: (0, 0))
_scl_spec = pl.BlockSpec((1, 1), lambda i: (0, 0))
_frow1_spec = pl.BlockSpec((FBLK, D), lambda i: (i + GRID, 0))


def _pre_body(x_ref, w_ref, deg0_ref, deg1_ref, hp_ref, dinv_ref):
  dinv = lax.rsqrt(deg0_ref[:, :1] + deg1_ref[:, :1] + 1.0)
  h = jnp.dot(x_ref[...], w_ref[...], preferred_element_type=jnp.float32)
  hp_ref[...] = dinv * h
  dinv_ref[...] = jnp.broadcast_to(dinv, (dinv.shape[0], DW))


def _ln_prelu(z, g_ref, be_ref, a_ref):
  mu = jnp.mean(z, axis=-1, keepdims=True)
  var = jnp.mean((z - mu) ** 2, axis=-1, keepdims=True)
  zn = (z - mu) * lax.rsqrt(var + 1e-5) * g_ref[...] + be_ref[...]
  return jnp.where(zn >= 0, zn, a_ref[0, 0] * zn)


def _mid_body(p0_ref, p1_ref, dinv_ref, b_ref, g_ref, be_ref,
              a_ref, w_ref, out_ref):
  dinv = dinv_ref[:, :1]
  z = dinv * (p0_ref[...] + p1_ref[...]) + b_ref[...]
  y = _ln_prelu(z, g_ref, be_ref, a_ref)
  out_ref[...] = dinv * jnp.dot(y, w_ref[...],
                                preferred_element_type=jnp.float32)


def _fin_body(p0_ref, p1_ref, dinv_ref, b_ref, g_ref, be_ref,
              a_ref, out_ref):
  dinv = dinv_ref[:, :1]
  z = dinv * (p0_ref[...] + p1_ref[...]) + b_ref[...]
  out_ref[...] = _ln_prelu(z, g_ref, be_ref, a_ref)


_pre_call = pl.pallas_call(
    _pre_body,
    grid=(GRID,),
    in_specs=[_row_spec, _mat_spec, _deg_spec, _deg1_spec],
    out_specs=[_row_spec, _dinv_spec],
    out_shape=[jax.ShapeDtypeStruct((NP, D), jnp.float32),
               jax.ShapeDtypeStruct((NP, DW), jnp.float32)],
)

_mid_call = pl.pallas_call(
    _mid_body,
    grid=(GRID,),
    in_specs=[_row_spec, _row1_spec, _dinv_spec,
              _vec_spec, _vec_spec, _vec_spec, _scl_spec, _mat_spec],
    out_specs=_row_spec,
    out_shape=jax.ShapeDtypeStruct((NP, D), jnp.float32),
)

_fin_call = pl.pallas_call(
    _fin_body,
    grid=(FGRID,),
    in_specs=[_row_spec, _frow1_spec, _dinv_spec,
              _vec_spec, _vec_spec, _vec_spec, _scl_spec],
    out_specs=_row_spec,
    out_shape=jax.ShapeDtypeStruct((NP, D), jnp.float32),
)


def kernel(x, edge_index, W1, b1, g1, be1, a1, W2, b2, g2, be2, a2):
  src = edge_index[0]
  dst = edge_index[1]
  ones_blk = jnp.ones((K, DEGW), jnp.float32)
  zeros_deg = jnp.zeros((NP, DEGW), jnp.float32)
  zerosd = jnp.zeros((NP, D), jnp.float32)

  degp = _deg_call(dst, ones_blk, zeros_deg)

  b1r, g1r, be1r = b1.reshape(1, D), g1.reshape(1, D), be1.reshape(1, D)
  b2r, g2r, be2r = b2.reshape(1, D), g2.reshape(1, D), be2.reshape(1, D)
  a1r, a2r = a1.reshape(1, 1), a2.reshape(1, 1)

  hp1, dinv8 = _pre_call(x, W1, degp, degp)
  parts1 = _msg_call(hp1, src, dst, zerosd)
  hp2 = _mid_call(parts1, parts1, dinv8, b1r, g1r, be1r, a1r, W2)
  parts2 = _msg_call(hp2, src, dst, zerosd)
  return _fin_call(parts2, parts2, dinv8, b2r, g2r, be2r, a2r)[:N]


# speedup vs baseline: 25.7672x; 1.0228x over previous
import functools

import jax
import jax.numpy as jnp
from jax import lax
from jax.experimental import pallas as pl
from jax.experimental.pallas import tpu as pltpu
from jax.experimental.pallas import tpu_sc as plsc

N = 10000
E = 320000
D = 128

NC = 2
NS = 16
NW = NC * NS
EPT = E // NW
K = 80
CH = EPT // K
NB = 4
GROUPS = CH // NB
TAIL = CH - GROUPS * NB
CPH = 2
NH = CH // CPH
MSG_TAIL = CH - NH * CPH
NP = 10240
RPT = NP // NS
DEGW = 128

_MESH = plsc.VectorSubcoreMesh(core_axis_name="c", subcore_axis_name="s")


@functools.partial(
    pl.kernel,
    out_type=jax.ShapeDtypeStruct((NC * NP, DEGW), jnp.float32),
    mesh=_MESH,
    scratch_types=(
        [pltpu.VMEM((NB, K), jnp.int32)]
        + [pltpu.VMEM((K, DEGW), jnp.float32)]
        + [pltpu.VMEM_SHARED((NP, DEGW), jnp.float32)]
        + [pltpu.SemaphoreType.DMA] * (NB + 1)
    ),
)
def _deg_call(dst_hbm, ones_hbm, zeros_hbm, out_hbm, *scr):
  idx_v = scr[0]
  ones_v = scr[1]
  acc_sh = scr[2]
  isem = scr[3:3 + NB]
  ssem = scr[3 + NB]
  cid = lax.axis_index("c")
  sid = lax.axis_index("s")
  wid = sid * NC + cid
  rbase = pl.multiple_of(sid * RPT, 8)

  pltpu.sync_copy(ones_hbm, ones_v)
  pltpu.sync_copy(zeros_hbm.at[pl.ds(rbase, RPT)], acc_sh.at[pl.ds(rbase, RPT)])
  plsc.subcore_barrier()

  def group(g, carry):
    @pl.when(g > 0)
    def _():
      for b in range(NB):
        pltpu.make_async_copy(ones_v, acc_sh.at[idx_v.at[b]], ssem).wait()

    gbase = wid * EPT + g * (NB * K)
    idesc = []
    for b in range(NB):
      base = pl.multiple_of(gbase + b * K, 8)
      idesc.append(
          pltpu.async_copy(dst_hbm.at[pl.ds(base, K)], idx_v.at[b], isem[b]))
    for b in range(NB):
      idesc[b].wait()
    for b in range(NB):
      pltpu.async_copy(ones_v, acc_sh.at[idx_v.at[b]], ssem, add=True)
    return carry

  lax.fori_loop(0, GROUPS, group, 0)
  for b in range(NB):
    pltpu.make_async_copy(ones_v, acc_sh.at[idx_v.at[b]], ssem).wait()
  for t in range(TAIL):
    base = pl.multiple_of(wid * EPT + (GROUPS * NB + t) * K, 8)
    pltpu.sync_copy(dst_hbm.at[pl.ds(base, K)], idx_v.at[0])
    pltpu.sync_copy(ones_v, acc_sh.at[idx_v.at[0]], add=True)
  plsc.subcore_barrier()

  obase = pl.multiple_of(cid * NP + sid * RPT, 8)
  pltpu.sync_copy(acc_sh.at[pl.ds(rbase, RPT)], out_hbm.at[pl.ds(obase, RPT)])


@functools.partial(
    pl.kernel,
    out_type=jax.ShapeDtypeStruct((NC * NP, D), jnp.float32),
    mesh=_MESH,
    scratch_types=(
        [pltpu.VMEM((2 * CPH * K,), jnp.int32)]
        + [pltpu.VMEM((2 * CPH, K), jnp.int32)]
        + [pltpu.VMEM((2 * CPH * K, D), jnp.float32)]
        + [pltpu.VMEM_SHARED((NP, D), jnp.float32)]
        + [pltpu.SemaphoreType.DMA] * 6
    ),
)
def _msg_call(hp_hbm, src_hbm, dst_hbm, zeros_hbm, out_hbm, *scr):
  sidx = scr[0]
  didx = scr[1]
  rows = scr[2]
  acc_sh = scr[3]
  isem = scr[4:6]
  gsem = scr[6:8]
  ssem = scr[8:10]
  cid = lax.axis_index("c")
  sid = lax.axis_index("s")
  wid = sid * NC + cid
  rbase = pl.multiple_of(sid * RPT, 8)

  @pl.when(cid == 0)
  def _():
    pltpu.sync_copy(hp_hbm.at[pl.ds(rbase, RPT)], acc_sh.at[pl.ds(rbase, RPT)])

  @pl.when(cid != 0)
  def _():
    pltpu.sync_copy(zeros_hbm.at[pl.ds(rbase, RPT)],
                    acc_sh.at[pl.ds(rbase, RPT)])

  plsc.subcore_barrier()

  def drain_scatters(slot):
    for b in range(CPH):
      off = slot * (CPH * K) + b * K
      pltpu.make_async_copy(rows.at[pl.ds(off, K)],
                            acc_sh.at[didx.at[CPH * slot + b]],
                            ssem[slot]).wait()

  def issue_half(h, slot):
    hbase = pl.multiple_of(wid * EPT + h * (CPH * K), 8)
    soff = slot * (CPH * K)
    descs = [pltpu.async_copy(src_hbm.at[pl.ds(hbase, CPH * K)],
                              sidx.at[pl.ds(soff, CPH * K)], isem[slot])]
    for b in range(CPH):
      base = pl.multiple_of(hbase + b * K, 8)
      descs.append(pltpu.async_copy(dst_hbm.at[pl.ds(base, K)],
                                    didx.at[CPH * slot + b], isem[slot]))
    for d in descs:
      d.wait()
    for b in range(CPH):
      pltpu.async_copy(hp_hbm.at[sidx.at[pl.ds(soff + b * K, K)]],
                       rows.at[pl.ds(soff + b * K, K)], gsem[slot])

  def drain_half(slot):
    for b in range(CPH):
      off = slot * (CPH * K) + b * K
      pltpu.make_async_copy(hp_hbm.at[sidx.at[pl.ds(off, K)]],
                            rows.at[pl.ds(off, K)], gsem[slot]).wait()

  issue_half(0, 0)

  def half_body(h, carry):
    slot = lax.rem(h, 2)

    @pl.when(jnp.logical_and(slot == 0, h + 1 < NH))
    def _():
      @pl.when(h >= 1)
      def _():
        drain_scatters(1)
      issue_half(h + 1, 1)

    @pl.when(jnp.logical_and(slot != 0, h + 1 < NH))
    def _():
      drain_scatters(0)
      issue_half(h + 1, 0)

    @pl.when(slot == 0)
    def _():
      drain_half(0)
      for b in range(CPH):
        pltpu.async_copy(rows.at[pl.ds(b * K, K)],
                         acc_sh.at[didx.at[b]], ssem[0], add=True)

    @pl.when(slot != 0)
    def _():
      drain_half(1)
      for b in range(CPH):
        off = CPH * K + b * K
        pltpu.async_copy(rows.at[pl.ds(off, K)],
                         acc_sh.at[didx.at[CPH + b]], ssem[1], add=True)

    return carry

  lax.fori_loop(0, NH, half_body, 0)
  drain_scatters(0)
  drain_scatters(1)

  for t in range(MSG_TAIL):
    base = pl.multiple_of(wid * EPT + (NH * CPH + t) * K, 8)
    pltpu.sync_copy(src_hbm.at[pl.ds(base, K)], sidx.at[pl.ds(0, K)])
    pltpu.sync_copy(dst_hbm.at[pl.ds(base, K)], didx.at[0])
    pltpu.async_copy(hp_hbm.at[sidx.at[pl.ds(0, K)]],
                     rows.at[pl.ds(0, K)], gsem[0]).wait()
    pltpu.sync_copy(rows.at[pl.ds(0, K)], acc_sh.at[didx.at[0]], add=True)
  plsc.subcore_barrier()

  obase = pl.multiple_of(cid * NP + sid * RPT, 8)
  pltpu.sync_copy(acc_sh.at[pl.ds(rbase, RPT)], out_hbm.at[pl.ds(obase, RPT)])


BLK = 1280
GRID = NP // BLK
FBLK = BLK
FGRID = GRID
DW = 8

_row_spec = pl.BlockSpec((BLK, D), lambda i: (i, 0))
_row1_spec = pl.BlockSpec((BLK, D), lambda i: (i + GRID, 0))
_deg_spec = pl.BlockSpec((BLK, DEGW), lambda i: (i, 0))
_deg1_spec = pl.BlockSpec((BLK, DEGW), lambda i: (i + GRID, 0))
_dinv_spec = pl.BlockSpec((BLK, DW), lambda i: (i, 0))
_mat_spec = pl.BlockSpec((D, D), lambda i: (0, 0))
_vec_spec = pl.BlockSpec((1, D), lambda i: (0, 0))
_scl_spec = pl.BlockSpec((1, 1), lambda i: (0, 0))
_frow1_spec = pl.BlockSpec((FBLK, D), lambda i: (i + GRID, 0))


def _pre_body(x_ref, w_ref, deg0_ref, deg1_ref, hp_ref, dinv_ref):
  dinv = lax.rsqrt(deg0_ref[:, :1] + deg1_ref[:, :1] + 1.0)
  h = jnp.dot(x_ref[...], w_ref[...], preferred_element_type=jnp.float32)
  hp_ref[...] = dinv * h
  dinv_ref[...] = jnp.broadcast_to(dinv, (dinv.shape[0], DW))


def _ln_prelu(z, g_ref, be_ref, a_ref):
  mu = jnp.mean(z, axis=-1, keepdims=True)
  var = jnp.mean((z - mu) ** 2, axis=-1, keepdims=True)
  zn = (z - mu) * lax.rsqrt(var + 1e-5) * g_ref[...] + be_ref[...]
  return jnp.where(zn >= 0, zn, a_ref[0, 0] * zn)


def _mid_body(p0_ref, p1_ref, dinv_ref, b_ref, g_ref, be_ref,
              a_ref, w_ref, out_ref):
  dinv = dinv_ref[:, :1]
  z = dinv * (p0_ref[...] + p1_ref[...]) + b_ref[...]
  y = _ln_prelu(z, g_ref, be_ref, a_ref)
  out_ref[...] = dinv * jnp.dot(y, w_ref[...],
                                preferred_element_type=jnp.float32)


def _fin_body(p0_ref, p1_ref, dinv_ref, b_ref, g_ref, be_ref,
              a_ref, out_ref):
  dinv = dinv_ref[:, :1]
  z = dinv * (p0_ref[...] + p1_ref[...]) + b_ref[...]
  out_ref[...] = _ln_prelu(z, g_ref, be_ref, a_ref)


_pre_call = pl.pallas_call(
    _pre_body,
    grid=(GRID,),
    in_specs=[_row_spec, _mat_spec, _deg_spec, _deg1_spec],
    out_specs=[_row_spec, _dinv_spec],
    out_shape=[jax.ShapeDtypeStruct((NP, D), jnp.float32),
               jax.ShapeDtypeStruct((NP, DW), jnp.float32)],
)

_mid_call = pl.pallas_call(
    _mid_body,
    grid=(GRID,),
    in_specs=[_row_spec, _row1_spec, _dinv_spec,
              _vec_spec, _vec_spec, _vec_spec, _scl_spec, _mat_spec],
    out_specs=_row_spec,
    out_shape=jax.ShapeDtypeStruct((NP, D), jnp.float32),
)

_fin_call = pl.pallas_call(
    _fin_body,
    grid=(FGRID,),
    in_specs=[_row_spec, _frow1_spec, _dinv_spec,
              _vec_spec, _vec_spec, _vec_spec, _scl_spec],
    out_specs=_row_spec,
    out_shape=jax.ShapeDtypeStruct((NP, D), jnp.float32),
)


def kernel(x, edge_index, W1, b1, g1, be1, a1, W2, b2, g2, be2, a2):
  src = edge_index[0]
  dst = edge_index[1]
  ones_blk = jnp.ones((K, DEGW), jnp.float32)
  zeros_deg = jnp.zeros((NP, DEGW), jnp.float32)
  zerosd = jnp.zeros((NP, D), jnp.float32)

  degp = _deg_call(dst, ones_blk, zeros_deg)

  b1r, g1r, be1r = b1.reshape(1, D), g1.reshape(1, D), be1.reshape(1, D)
  b2r, g2r, be2r = b2.reshape(1, D), g2.reshape(1, D), be2.reshape(1, D)
  a1r, a2r = a1.reshape(1, 1), a2.reshape(1, 1)

  hp1, dinv8 = _pre_call(x, W1, degp, degp)
  parts1 = _msg_call(hp1, src, dst, zerosd)
  hp2 = _mid_call(parts1, parts1, dinv8, b1r, g1r, be1r, a1r, W2)
  parts2 = _msg_call(hp2, src, dst, zerosd)
  return _fin_call(parts2, parts2, dinv8, b2r, g2r, be2r, a2r)[:N]


# speedup vs baseline: 25.8895x; 1.0047x over previous
import functools

import jax
import jax.numpy as jnp
from jax import lax
from jax.experimental import pallas as pl
from jax.experimental.pallas import tpu as pltpu
from jax.experimental.pallas import tpu_sc as plsc

N = 10000
E = 320000
D = 128

NC = 2
NS = 16
NW = NC * NS
EPT = E // NW
K = 80
CH = EPT // K
NB = 8
GROUPS = CH // NB
TAIL = CH - GROUPS * NB
CPH = 2
NH = CH // CPH
MSG_TAIL = CH - NH * CPH
NP = 10240
RPT = NP // NS
DEGW = 128

_MESH = plsc.VectorSubcoreMesh(core_axis_name="c", subcore_axis_name="s")


@functools.partial(
    pl.kernel,
    out_type=jax.ShapeDtypeStruct((NC * NP, DEGW), jnp.float32),
    mesh=_MESH,
    scratch_types=(
        [pltpu.VMEM((NB, K), jnp.int32)]
        + [pltpu.VMEM((K, DEGW), jnp.float32)]
        + [pltpu.VMEM_SHARED((NP, DEGW), jnp.float32)]
        + [pltpu.SemaphoreType.DMA] * (NB + 1)
    ),
)
def _deg_call(dst_hbm, ones_hbm, zeros_hbm, out_hbm, *scr):
  idx_v = scr[0]
  ones_v = scr[1]
  acc_sh = scr[2]
  isem = scr[3:3 + NB]
  ssem = scr[3 + NB]
  cid = lax.axis_index("c")
  sid = lax.axis_index("s")
  wid = sid * NC + cid
  rbase = pl.multiple_of(sid * RPT, 8)

  pltpu.sync_copy(ones_hbm, ones_v)
  pltpu.sync_copy(zeros_hbm.at[pl.ds(rbase, RPT)], acc_sh.at[pl.ds(rbase, RPT)])
  plsc.subcore_barrier()

  def group(g, carry):
    @pl.when(g > 0)
    def _():
      for b in range(NB):
        pltpu.make_async_copy(ones_v, acc_sh.at[idx_v.at[b]], ssem).wait()

    gbase = wid * EPT + g * (NB * K)
    idesc = []
    for b in range(NB):
      base = pl.multiple_of(gbase + b * K, 8)
      idesc.append(
          pltpu.async_copy(dst_hbm.at[pl.ds(base, K)], idx_v.at[b], isem[b]))
    for b in range(NB):
      idesc[b].wait()
    for b in range(NB):
      pltpu.async_copy(ones_v, acc_sh.at[idx_v.at[b]], ssem, add=True)
    return carry

  lax.fori_loop(0, GROUPS, group, 0)
  for b in range(NB):
    pltpu.make_async_copy(ones_v, acc_sh.at[idx_v.at[b]], ssem).wait()
  for t in range(TAIL):
    base = pl.multiple_of(wid * EPT + (GROUPS * NB + t) * K, 8)
    pltpu.sync_copy(dst_hbm.at[pl.ds(base, K)], idx_v.at[0])
    pltpu.sync_copy(ones_v, acc_sh.at[idx_v.at[0]], add=True)
  plsc.subcore_barrier()

  obase = pl.multiple_of(cid * NP + sid * RPT, 8)
  pltpu.sync_copy(acc_sh.at[pl.ds(rbase, RPT)], out_hbm.at[pl.ds(obase, RPT)])


@functools.partial(
    pl.kernel,
    out_type=jax.ShapeDtypeStruct((NC * NP, D), jnp.float32),
    mesh=_MESH,
    scratch_types=(
        [pltpu.VMEM((2 * CPH * K,), jnp.int32)]
        + [pltpu.VMEM((2 * CPH, K), jnp.int32)]
        + [pltpu.VMEM((2 * CPH * K, D), jnp.float32)]
        + [pltpu.VMEM_SHARED((NP, D), jnp.float32)]
        + [pltpu.SemaphoreType.DMA] * 6
    ),
)
def _msg_call(hp_hbm, src_hbm, dst_hbm, zeros_hbm, out_hbm, *scr):
  sidx = scr[0]
  didx = scr[1]
  rows = scr[2]
  acc_sh = scr[3]
  isem = scr[4:6]
  gsem = scr[6:8]
  ssem = scr[8:10]
  cid = lax.axis_index("c")
  sid = lax.axis_index("s")
  wid = sid * NC + cid
  rbase = pl.multiple_of(sid * RPT, 8)

  @pl.when(cid == 0)
  def _():
    pltpu.sync_copy(hp_hbm.at[pl.ds(rbase, RPT)], acc_sh.at[pl.ds(rbase, RPT)])

  @pl.when(cid != 0)
  def _():
    pltpu.sync_copy(zeros_hbm.at[pl.ds(rbase, RPT)],
                    acc_sh.at[pl.ds(rbase, RPT)])

  plsc.subcore_barrier()

  def drain_scatters(slot):
    for b in range(CPH):
      off = slot * (CPH * K) + b * K
      pltpu.make_async_copy(rows.at[pl.ds(off, K)],
                            acc_sh.at[didx.at[CPH * slot + b]],
                            ssem[slot]).wait()

  def issue_half(h, slot):
    hbase = pl.multiple_of(wid * EPT + h * (CPH * K), 8)
    soff = slot * (CPH * K)
    descs = [pltpu.async_copy(src_hbm.at[pl.ds(hbase, CPH * K)],
                              sidx.at[pl.ds(soff, CPH * K)], isem[slot])]
    for b in range(CPH):
      base = pl.multiple_of(hbase + b * K, 8)
      descs.append(pltpu.async_copy(dst_hbm.at[pl.ds(base, K)],
                                    didx.at[CPH * slot + b], isem[slot]))
    for d in descs:
      d.wait()
    for b in range(CPH):
      pltpu.async_copy(hp_hbm.at[sidx.at[pl.ds(soff + b * K, K)]],
                       rows.at[pl.ds(soff + b * K, K)], gsem[slot])

  def drain_half(slot):
    for b in range(CPH):
      off = slot * (CPH * K) + b * K
      pltpu.make_async_copy(hp_hbm.at[sidx.at[pl.ds(off, K)]],
                            rows.at[pl.ds(off, K)], gsem[slot]).wait()

  issue_half(0, 0)

  def half_body(h, carry):
    slot = lax.rem(h, 2)

    @pl.when(jnp.logical_and(slot == 0, h + 1 < NH))
    def _():
      @pl.when(h >= 1)
      def _():
        drain_scatters(1)
      issue_half(h + 1, 1)

    @pl.when(jnp.logical_and(slot != 0, h + 1 < NH))
    def _():
      drain_scatters(0)
      issue_half(h + 1, 0)

    @pl.when(slot == 0)
    def _():
      drain_half(0)
      for b in range(CPH):
        pltpu.async_copy(rows.at[pl.ds(b * K, K)],
                         acc_sh.at[didx.at[b]], ssem[0], add=True)

    @pl.when(slot != 0)
    def _():
      drain_half(1)
      for b in range(CPH):
        off = CPH * K + b * K
        pltpu.async_copy(rows.at[pl.ds(off, K)],
                         acc_sh.at[didx.at[CPH + b]], ssem[1], add=True)

    return carry

  lax.fori_loop(0, NH, half_body, 0)
  drain_scatters(0)
  drain_scatters(1)

  for t in range(MSG_TAIL):
    base = pl.multiple_of(wid * EPT + (NH * CPH + t) * K, 8)
    pltpu.sync_copy(src_hbm.at[pl.ds(base, K)], sidx.at[pl.ds(0, K)])
    pltpu.sync_copy(dst_hbm.at[pl.ds(base, K)], didx.at[0])
    pltpu.async_copy(hp_hbm.at[sidx.at[pl.ds(0, K)]],
                     rows.at[pl.ds(0, K)], gsem[0]).wait()
    pltpu.sync_copy(rows.at[pl.ds(0, K)], acc_sh.at[didx.at[0]], add=True)
  plsc.subcore_barrier()

  obase = pl.multiple_of(cid * NP + sid * RPT, 8)
  pltpu.sync_copy(acc_sh.at[pl.ds(rbase, RPT)], out_hbm.at[pl.ds(obase, RPT)])


BLK = 1280
GRID = NP // BLK
FBLK = BLK
FGRID = GRID
DW = 8

_row_spec = pl.BlockSpec((BLK, D), lambda i: (i, 0))
_row1_spec = pl.BlockSpec((BLK, D), lambda i: (i + GRID, 0))
_deg_spec = pl.BlockSpec((BLK, DEGW), lambda i: (i, 0))
_deg1_spec = pl.BlockSpec((BLK, DEGW), lambda i: (i + GRID, 0))
_dinv_spec = pl.BlockSpec((BLK, DW), lambda i: (i, 0))
_mat_spec = pl.BlockSpec((D, D), lambda i: (0, 0))
_vec_spec = pl.BlockSpec((1, D), lambda i: (0, 0))
_scl_spec = pl.BlockSpec((1, 1), lambda i: (0, 0))
_frow1_spec = pl.BlockSpec((FBLK, D), lambda i: (i + GRID, 0))


def _pre_body(x_ref, w_ref, deg0_ref, deg1_ref, hp_ref, dinv_ref):
  dinv = lax.rsqrt(deg0_ref[:, :1] + deg1_ref[:, :1] + 1.0)
  h = jnp.dot(x_ref[...], w_ref[...], preferred_element_type=jnp.float32)
  hp_ref[...] = dinv * h
  dinv_ref[...] = jnp.broadcast_to(dinv, (dinv.shape[0], DW))


def _ln_prelu(z, g_ref, be_ref, a_ref):
  mu = jnp.mean(z, axis=-1, keepdims=True)
  var = jnp.mean((z - mu) ** 2, axis=-1, keepdims=True)
  zn = (z - mu) * lax.rsqrt(var + 1e-5) * g_ref[...] + be_ref[...]
  return jnp.where(zn >= 0, zn, a_ref[0, 0] * zn)


def _mid_body(p0_ref, p1_ref, dinv_ref, b_ref, g_ref, be_ref,
              a_ref, w_ref, out_ref):
  dinv = dinv_ref[:, :1]
  z = dinv * (p0_ref[...] + p1_ref[...]) + b_ref[...]
  y = _ln_prelu(z, g_ref, be_ref, a_ref)
  out_ref[...] = dinv * jnp.dot(y, w_ref[...],
                                preferred_element_type=jnp.float32)


def _fin_body(p0_ref, p1_ref, dinv_ref, b_ref, g_ref, be_ref,
              a_ref, out_ref):
  dinv = dinv_ref[:, :1]
  z = dinv * (p0_ref[...] + p1_ref[...]) + b_ref[...]
  out_ref[...] = _ln_prelu(z, g_ref, be_ref, a_ref)


_pre_call = pl.pallas_call(
    _pre_body,
    grid=(GRID,),
    in_specs=[_row_spec, _mat_spec, _deg_spec, _deg1_spec],
    out_specs=[_row_spec, _dinv_spec],
    out_shape=[jax.ShapeDtypeStruct((NP, D), jnp.float32),
               jax.ShapeDtypeStruct((NP, DW), jnp.float32)],
)

_mid_call = pl.pallas_call(
    _mid_body,
    grid=(GRID,),
    in_specs=[_row_spec, _row1_spec, _dinv_spec,
              _vec_spec, _vec_spec, _vec_spec, _scl_spec, _mat_spec],
    out_specs=_row_spec,
    out_shape=jax.ShapeDtypeStruct((NP, D), jnp.float32),
)

_fin_call = pl.pallas_call(
    _fin_body,
    grid=(FGRID,),
    in_specs=[_row_spec, _frow1_spec, _dinv_spec,
              _vec_spec, _vec_spec, _vec_spec, _scl_spec],
    out_specs=_row_spec,
    out_shape=jax.ShapeDtypeStruct((NP, D), jnp.float32),
)


def kernel(x, edge_index, W1, b1, g1, be1, a1, W2, b2, g2, be2, a2):
  src = edge_index[0]
  dst = edge_index[1]
  ones_blk = jnp.ones((K, DEGW), jnp.float32)
  zeros_deg = jnp.zeros((NP, DEGW), jnp.float32)
  zerosd = jnp.zeros((NP, D), jnp.float32)

  degp = _deg_call(dst, ones_blk, zeros_deg)

  b1r, g1r, be1r = b1.reshape(1, D), g1.reshape(1, D), be1.reshape(1, D)
  b2r, g2r, be2r = b2.reshape(1, D), g2.reshape(1, D), be2.reshape(1, D)
  a1r, a2r = a1.reshape(1, 1), a2.reshape(1, 1)

  hp1, dinv8 = _pre_call(x, W1, degp, degp)
  parts1 = _msg_call(hp1, src, dst, zerosd)
  hp2 = _mid_call(parts1, parts1, dinv8, b1r, g1r, be1r, a1r, W2)
  parts2 = _msg_call(hp2, src, dst, zerosd)
  return _fin_call(parts2, parts2, dinv8, b2r, g2r, be2r, a2r)[:N]
